# Initial kernel scaffold; baseline (speedup 1.0000x reference)
import jax
import jax.numpy as jnp
from jax.experimental import pallas as pl


def kernel(x, adj, W, bias):
    raise NotImplementedError("write your pallas kernel here")


# speedup vs baseline: 1.0240x; 1.0240x over previous
import jax
import jax.numpy as jnp
from jax.experimental import pallas as pl
from jax.experimental.pallas import tpu as pltpu

_BM = 400


def _gcn_block(x_ref, wt_ref, adj_ref, bias_ref, out_ref, s_ref):
    m = pl.program_id(0)

    @pl.when(m == 0)
    def _compute_support():
        s_ref[...] = jnp.dot(
            x_ref[...], wt_ref[...], preferred_element_type=jnp.float32
        )

    out_ref[...] = (
        jnp.dot(adj_ref[...], s_ref[...], preferred_element_type=jnp.float32)
        + bias_ref[...]
    )


def kernel(x, adj, W, bias):
    n, d_in = x.shape
    d_out = W.shape[0]
    wt = W.T
    bias2d = bias.reshape(1, d_out)
    return pl.pallas_call(
        _gcn_block,
        grid=(n // _BM,),
        in_specs=[
            pl.BlockSpec((n, d_in), lambda m: (0, 0)),
            pl.BlockSpec((d_in, d_out), lambda m: (0, 0)),
            pl.BlockSpec((_BM, n), lambda m: (m, 0)),
            pl.BlockSpec((1, d_out), lambda m: (0, 0)),
        ],
        out_specs=pl.BlockSpec((_BM, d_out), lambda m: (m, 0)),
        out_shape=jax.ShapeDtypeStruct((n, d_out), jnp.float32),
        scratch_shapes=[pltpu.VMEM((n, d_out), jnp.float32)],
        compiler_params=pltpu.CompilerParams(
            dimension_semantics=("arbitrary",),
        ),
    )(x, wt, adj, bias2d)


# speedup vs baseline: 1.0247x; 1.0007x over previous
import jax
import jax.numpy as jnp
from jax.experimental import pallas as pl
from jax.experimental.pallas import tpu as pltpu

_BM = 200


def _gcn_block(x_ref, wt_ref, adj_ref, bias_ref, out_ref, s_ref):
    m = pl.program_id(0)

    @pl.when(m == 0)
    def _compute_support():
        s_ref[...] = jnp.dot(
            x_ref[...], wt_ref[...], preferred_element_type=jnp.float32
        )

    out_ref[...] = (
        jnp.dot(adj_ref[...], s_ref[...], preferred_element_type=jnp.float32)
        + bias_ref[...]
    )


def kernel(x, adj, W, bias):
    n, d_in = x.shape
    d_out = W.shape[0]
    wt = W.T
    bias2d = bias.reshape(1, d_out)
    return pl.pallas_call(
        _gcn_block,
        grid=(n // _BM,),
        in_specs=[
            pl.BlockSpec((n, d_in), lambda m: (0, 0)),
            pl.BlockSpec((d_in, d_out), lambda m: (0, 0)),
            pl.BlockSpec((_BM, n), lambda m: (m, 0)),
            pl.BlockSpec((1, d_out), lambda m: (0, 0)),
        ],
        out_specs=pl.BlockSpec((_BM, d_out), lambda m: (m, 0)),
        out_shape=jax.ShapeDtypeStruct((n, d_out), jnp.float32),
        scratch_shapes=[pltpu.VMEM((n, d_out), jnp.float32)],
        compiler_params=pltpu.CompilerParams(
            dimension_semantics=("arbitrary",),
        ),
    )(x, wt, adj, bias2d)
